# baseline (speedup 1.0000x reference)
import jax
import jax.numpy as jnp
from jax import lax
from jax.experimental import pallas as pl
from jax.experimental.pallas import tpu as pltpu
from jax.experimental.pallas import tpu_sc as plsc

N = 10000
E = 320000
D = 128

NP = 10240
NC = 2
NS = 16
HALF = NP // NC
HP = 5248
DEN_P = HALF + 64
CH = 128
NE_REAL = E + N
NCH = (-(-NE_REAL // (NS * CH)) + 7) // 8 * 8
EP = NS * NCH * CH
WB = HALF // NS
PCAP = 384
BR = 1024

_f32 = jnp.float32


def _k1_body(x_ref, w_ref, att_ref, h_ref, asad_ref):
    h = jnp.dot(x_ref[...], w_ref[...], preferred_element_type=_f32)
    h_ref[...] = h
    asad_ref[...] = lax.dot_general(
        att_ref[...], h, (((1,), (1,)), ((), ())), preferred_element_type=_f32)


def _k2_body(un_ref, den_ref, b_ref, w_ref, att_ref, h_ref, asad_ref):
    u = un_ref[0]
    d = jnp.sum(den_ref[0], axis=0) + 1e-30
    h2 = jnp.maximum(u / d[:, None] + b_ref[...], 0.0)
    g = jnp.dot(h2, w_ref[...], preferred_element_type=_f32)
    h_ref[...] = g
    asad_ref[...] = lax.dot_general(
        att_ref[...], g, (((1,), (1,)), ((), ())), preferred_element_type=_f32)


def _k3_body(un_ref, den_ref, b_ref, out_ref):
    u = un_ref[0]
    d = jnp.sum(den_ref[0], axis=0) + 1e-30
    out_ref[...] = u / d[:, None] + b_ref[...]


_HB = HALF // BR

_k1_call = pl.pallas_call(
    _k1_body,
    grid=(NP // BR,),
    in_specs=[
        pl.BlockSpec((BR, D), lambda i: (i, 0)),
        pl.BlockSpec((D, D), lambda i: (0, 0)),
        pl.BlockSpec((8, D), lambda i: (0, 0)),
    ],
    out_specs=[
        pl.BlockSpec((BR, D), lambda i: (i, 0)),
        pl.BlockSpec((8, BR), lambda i: (0, i)),
    ],
    out_shape=[
        jax.ShapeDtypeStruct((NP, D), _f32),
        jax.ShapeDtypeStruct((8, NP), _f32),
    ],
)

_k2_call = pl.pallas_call(
    _k2_body,
    grid=(NP // BR,),
    in_specs=[
        pl.BlockSpec((1, BR, D), lambda i: (i // _HB, i % _HB, 0)),
        pl.BlockSpec((1, NS, BR), lambda i: (i // _HB, 0, i % _HB)),
        pl.BlockSpec((1, D), lambda i: (0, 0)),
        pl.BlockSpec((D, D), lambda i: (0, 0)),
        pl.BlockSpec((8, D), lambda i: (0, 0)),
    ],
    out_specs=[
        pl.BlockSpec((BR, D), lambda i: (i, 0)),
        pl.BlockSpec((8, BR), lambda i: (0, i)),
    ],
    out_shape=[
        jax.ShapeDtypeStruct((NP, D), _f32),
        jax.ShapeDtypeStruct((8, NP), _f32),
    ],
)

_k3_call = pl.pallas_call(
    _k3_body,
    grid=(NP // BR,),
    in_specs=[
        pl.BlockSpec((1, BR, D), lambda i: (i // _HB, i % _HB, 0)),
        pl.BlockSpec((1, NS, BR), lambda i: (i // _HB, 0, i % _HB)),
        pl.BlockSpec((1, D), lambda i: (0, 0)),
    ],
    out_specs=pl.BlockSpec((BR, D), lambda i: (i, 0)),
    out_shape=jax.ShapeDtypeStruct((NP, D), _f32),
)


def _sc_body(h_hbm, as_hbm, ad_hbm, src_hbm, dst_hbm, un_out, den_out,
             as_t, ad_t, den_t, src8, dst8, dlc_t, rows_a, rows_s,
             psrc, pw, pdl, psnap, pwsnap, out_sh, gsem_a, ssem_a):
    cid = lax.axis_index("c")
    sid = lax.axis_index("s")

    z16 = jnp.zeros((16,), _f32)

    def _zrow(r, _):
        for k in range(D // 16):
            rows_a[r, pl.ds(k * 16, 16)] = z16
        return _
    lax.fori_loop(0, CH, _zrow, None)
    for q in range(-(-(HP // NS)) // CH + 1):
        row0 = jnp.minimum(sid * (HP // NS) + q * CH, HP - CH)
        pltpu.sync_copy(rows_a, out_sh.at[pl.ds(row0, CH)])

    def _zden(i, _):
        den_t[pl.ds(i * 16, 16)] = z16
        return _
    lax.fori_loop(0, DEN_P // 16, _zden, None)

    pltpu.sync_copy(as_hbm, as_t)
    pltpu.sync_copy(ad_hbm, ad_t)

    def _zrow2(r, _):
        for k in range(D // 16):
            rows_s[r, pl.ds(k * 16, 16)] = z16
        return _
    lax.fori_loop(0, CH, _zrow2, None)
    for j in range(CH // 16):
        pwsnap[pl.ds(j * 16, 16)] = z16
        psnap[pl.ds(j * 16, 16)] = jnp.zeros((16,), jnp.int32)
        dlc_t[0, pl.ds(j * 16, 16)] = jnp.zeros((16,), jnp.int32)

    pltpu.async_copy(rows_s, out_sh.at[pl.ds(HALF, CH)], ssem_a)
    pltpu.async_copy(h_hbm.at[psnap.at[pl.ds(0, CH)]], rows_a, gsem_a)

    plsc.subcore_barrier()

    base = cid * HALF

    def _drain():
        pltpu.make_async_copy(h_hbm.at[pl.ds(0, CH)], rows_s, ssem_a).wait()

    def _complete():
        _drain()
        pltpu.make_async_copy(h_hbm.at[pl.ds(0, CH)], rows_a, gsem_a).wait()

        def _mrow(j, _):
            w16 = pwsnap[pl.ds(j * 16, 16)]
            for l in range(16):
                wv = w16[l]
                e = j * 16 + l
                for k in range(D // 16):
                    rows_s[e, pl.ds(k * 16, 16)] = (
                        rows_a[e, pl.ds(k * 16, 16)] * wv)
            return _
        lax.fori_loop(0, CH // 16, _mrow, None)
        for j in range(CH // 16):
            dlc_t[1, pl.ds(j * 16, 16)] = dlc_t[0, pl.ds(j * 16, 16)]
        pltpu.async_copy(rows_s, out_sh.at[dlc_t.at[1]], ssem_a, add=True)

    def _flush_shift(q):
        _complete()
        for j in range(CH // 16):
            psnap[pl.ds(j * 16, 16)] = psrc[pl.ds(j * 16, 16)]
            pwsnap[pl.ds(j * 16, 16)] = pw[pl.ds(j * 16, 16)]
            dlc_t[0, pl.ds(j * 16, 16)] = pdl[pl.ds(j * 16, 16)]
        pltpu.async_copy(h_hbm.at[psnap.at[pl.ds(0, CH)]], rows_a, gsem_a)
        for j in range(CH // 16):
            psrc[pl.ds(j * 16, 16)] = psrc[pl.ds(CH + j * 16, 16)]
            pw[pl.ds(j * 16, 16)] = pw[pl.ds(CH + j * 16, 16)]
            pdl[pl.ds(j * 16, 16)] = pdl[pl.ds(CH + j * 16, 16)]
        return q - CH

    def _super(s, qc):
        row0 = sid * NCH + s * 8
        pltpu.sync_copy(src_hbm.at[pl.ds(row0, 8)], src8)
        pltpu.sync_copy(dst_hbm.at[pl.ds(row0, 8)], dst8)

        def _chunk(jj, qc):
            for j in range(CH // 16):
                s16 = src8[jj, pl.ds(j * 16, 16)]
                d16 = dst8[jj, pl.ds(j * 16, 16)]
                a = (plsc.load_gather(as_t, [s16])
                     + plsc.load_gather(ad_t, [d16]))
                a = jnp.where(a > 0, a, a * jnp.float32(0.2))
                w = jnp.exp(a)
                dl = d16 - base
                ok = (dl >= 0) & (dl < HALF)
                plsc.addupdate_scatter(
                    den_t, [jnp.where(ok, dl, jnp.int32(HALF))], w)
                plsc.store_compressed(psrc.at[pl.ds(qc, 16)], s16, mask=ok)
                plsc.store_compressed(pw.at[pl.ds(qc, 16)], w, mask=ok)
                plsc.store_compressed(pdl.at[pl.ds(qc, 16)], dl, mask=ok)
                qc = qc + plsc.all_reduce_population_count(ok)[0]
            return lax.while_loop(lambda q: q >= CH, _flush_shift, qc)

        return lax.fori_loop(0, 8, _chunk, qc)

    qc = lax.fori_loop(0, NCH // 8, _super, jnp.int32(0))

    iota16 = lax.iota(jnp.int32, 16)
    for j in range(CH // 16):
        m = (iota16 + (j * 16)) >= qc
        psrc[pl.ds(j * 16, 16)] = jnp.where(
            m, jnp.int32(NP - 1), psrc[pl.ds(j * 16, 16)])
        pw[pl.ds(j * 16, 16)] = jnp.where(
            m, jnp.float32(0.0), pw[pl.ds(j * 16, 16)])
        pdl[pl.ds(j * 16, 16)] = jnp.where(
            m, jnp.int32(HALF), pdl[pl.ds(j * 16, 16)])
    _flush_shift(jnp.int32(CH))
    _complete()
    _drain()

    plsc.subcore_barrier()

    pltpu.sync_copy(den_t.at[pl.ds(0, HALF)],
                    den_out.at[pl.ds((cid * NS + sid) * HALF, HALF)])
    pltpu.sync_copy(out_sh.at[pl.ds(sid * WB, WB)],
                    un_out.at[cid, pl.ds(sid * WB, WB)])


_sc_call = pl.kernel(
    _sc_body,
    out_type=[
        jax.ShapeDtypeStruct((NC, HALF, D), _f32),
        jax.ShapeDtypeStruct((NC * NS * HALF,), _f32),
    ],
    mesh=plsc.VectorSubcoreMesh(
        core_axis_name="c", subcore_axis_name="s",
        num_cores=NC, num_subcores=NS),
    compiler_params=pltpu.CompilerParams(needs_layout_passes=False),
    scratch_types=[
        pltpu.VMEM((NP,), _f32),
        pltpu.VMEM((NP,), _f32),
        pltpu.VMEM((DEN_P,), _f32),
        pltpu.VMEM((8, CH), jnp.int32),
        pltpu.VMEM((8, CH), jnp.int32),
        pltpu.VMEM((8, CH), jnp.int32),
        pltpu.VMEM((CH, D), _f32),
        pltpu.VMEM((CH, D), _f32),
        pltpu.VMEM((PCAP,), jnp.int32),
        pltpu.VMEM((PCAP,), _f32),
        pltpu.VMEM((PCAP,), jnp.int32),
        pltpu.VMEM((CH,), jnp.int32),
        pltpu.VMEM((CH,), _f32),
        pltpu.VMEM_SHARED((HP, D), _f32),
        pltpu.SemaphoreType.DMA,
        pltpu.SemaphoreType.DMA,
    ],
)


@jax.jit
def kernel(x, edge_index, W1, att_src1, att_dst1, bias1,
           W2, att_src2, att_dst2, bias2):
    x_pad = jnp.zeros((NP, D), _f32).at[:N].set(x)
    ei = edge_index.astype(jnp.int32)
    loop = jnp.arange(N, dtype=jnp.int32)
    pad = jnp.full((EP - NE_REAL,), NP - 1, jnp.int32)
    src = jnp.concatenate([ei[0], loop, pad]).reshape(NS * NCH, CH)
    dst = jnp.concatenate([ei[1], loop, pad]).reshape(NS * NCH, CH)
    att1 = jnp.zeros((8, D), _f32).at[0].set(att_src1).at[1].set(att_dst1)
    att2 = jnp.zeros((8, D), _f32).at[0].set(att_src2).at[1].set(att_dst2)
    b1 = bias1.reshape(1, D)
    b2 = bias2.reshape(1, D)

    h1, asad1 = _k1_call(x_pad, W1, att1)
    un1, den1 = _sc_call(h1, asad1[0], asad1[1], src, dst)
    g2, asad2 = _k2_call(un1, den1.reshape(NC, NS, HALF), b1, W2, att2)
    un2, den2 = _sc_call(g2, asad2[0], asad2[1], src, dst)
    out = _k3_call(un2, den2.reshape(NC, NS, HALF), b2)
    return out[:N]


# speedup vs baseline: 1.0711x; 1.0711x over previous
import jax
import jax.numpy as jnp
from jax import lax
from jax.experimental import pallas as pl
from jax.experimental.pallas import tpu as pltpu
from jax.experimental.pallas import tpu_sc as plsc

N = 10000
E = 320000
D = 128

NP = 10240
NC = 2
NS = 16
HALF = NP // NC
HP = 5248
DEN_P = HALF + 64
CH = 128
NE_REAL = E + N
NCH = (-(-NE_REAL // (NS * CH)) + 7) // 8 * 8
EP = NS * NCH * CH
WB = HALF // NS
PCAP = 384
BR = 1024

_f32 = jnp.float32


def _k1_body(x_ref, w_ref, att_ref, h_ref, asad_ref):
    h = jnp.dot(x_ref[...], w_ref[...], preferred_element_type=_f32)
    h_ref[...] = h
    asad_ref[...] = lax.dot_general(
        att_ref[...], h, (((1,), (1,)), ((), ())), preferred_element_type=_f32)


def _k2_body(un_ref, den_ref, b_ref, w_ref, att_ref, h_ref, asad_ref):
    u = un_ref[0]
    d = jnp.sum(den_ref[0], axis=0) + 1e-30
    h2 = jnp.maximum(u / d[:, None] + b_ref[...], 0.0)
    g = jnp.dot(h2, w_ref[...], preferred_element_type=_f32)
    h_ref[...] = g
    asad_ref[...] = lax.dot_general(
        att_ref[...], g, (((1,), (1,)), ((), ())), preferred_element_type=_f32)


def _k3_body(un_ref, den_ref, b_ref, out_ref):
    u = un_ref[0]
    d = jnp.sum(den_ref[0], axis=0) + 1e-30
    out_ref[...] = u / d[:, None] + b_ref[...]


_HB = HALF // BR

_k1_call = pl.pallas_call(
    _k1_body,
    grid=(NP // BR,),
    in_specs=[
        pl.BlockSpec((BR, D), lambda i: (i, 0)),
        pl.BlockSpec((D, D), lambda i: (0, 0)),
        pl.BlockSpec((8, D), lambda i: (0, 0)),
    ],
    out_specs=[
        pl.BlockSpec((BR, D), lambda i: (i, 0)),
        pl.BlockSpec((8, BR), lambda i: (0, i)),
    ],
    out_shape=[
        jax.ShapeDtypeStruct((NP, D), _f32),
        jax.ShapeDtypeStruct((8, NP), _f32),
    ],
)

_k2_call = pl.pallas_call(
    _k2_body,
    grid=(NP // BR,),
    in_specs=[
        pl.BlockSpec((1, BR, D), lambda i: (i // _HB, i % _HB, 0)),
        pl.BlockSpec((1, NS, BR), lambda i: (i // _HB, 0, i % _HB)),
        pl.BlockSpec((1, D), lambda i: (0, 0)),
        pl.BlockSpec((D, D), lambda i: (0, 0)),
        pl.BlockSpec((8, D), lambda i: (0, 0)),
    ],
    out_specs=[
        pl.BlockSpec((BR, D), lambda i: (i, 0)),
        pl.BlockSpec((8, BR), lambda i: (0, i)),
    ],
    out_shape=[
        jax.ShapeDtypeStruct((NP, D), _f32),
        jax.ShapeDtypeStruct((8, NP), _f32),
    ],
)

_k3_call = pl.pallas_call(
    _k3_body,
    grid=(NP // BR,),
    in_specs=[
        pl.BlockSpec((1, BR, D), lambda i: (i // _HB, i % _HB, 0)),
        pl.BlockSpec((1, NS, BR), lambda i: (i // _HB, 0, i % _HB)),
        pl.BlockSpec((1, D), lambda i: (0, 0)),
    ],
    out_specs=pl.BlockSpec((BR, D), lambda i: (i, 0)),
    out_shape=jax.ShapeDtypeStruct((NP, D), _f32),
)


def _sc_body(h_hbm, as_hbm, ad_hbm, src_hbm, dst_hbm, un_out, den_out,
             as_t, ad_t, den_t, src8, dst8, dlc_t, rows_a, rows_s,
             psrc, pw, pdl, psnap, pwsnap, out_sh, gsem_a, ssem_a):
    cid = lax.axis_index("c")
    sid = lax.axis_index("s")

    z16 = jnp.zeros((16,), _f32)

    def _zrow(r, _):
        for k in range(D // 16):
            rows_a[r, pl.ds(k * 16, 16)] = z16
        return _
    lax.fori_loop(0, CH, _zrow, None)
    for q in range(-(-(HP // NS)) // CH + 1):
        row0 = jnp.minimum(sid * (HP // NS) + q * CH, HP - CH)
        pltpu.sync_copy(rows_a, out_sh.at[pl.ds(row0, CH)])

    def _zden(i, _):
        den_t[pl.ds(i * 16, 16)] = z16
        return _
    lax.fori_loop(0, DEN_P // 16, _zden, None)

    pltpu.sync_copy(as_hbm, as_t)
    pltpu.sync_copy(ad_hbm, ad_t)

    def _zrow2(r, _):
        for k in range(D // 16):
            rows_s[r, pl.ds(k * 16, 16)] = z16
        return _
    lax.fori_loop(0, CH, _zrow2, None)
    for j in range(CH // 16):
        pwsnap[pl.ds(j * 16, 16)] = z16
        psnap[pl.ds(j * 16, 16)] = jnp.zeros((16,), jnp.int32)
        dlc_t[0, pl.ds(j * 16, 16)] = jnp.zeros((16,), jnp.int32)

    pltpu.async_copy(rows_s, out_sh.at[pl.ds(HALF, CH)], ssem_a)
    pltpu.async_copy(h_hbm.at[psnap.at[pl.ds(0, CH)]], rows_a, gsem_a)

    plsc.subcore_barrier()

    base = cid * HALF

    def _drain():
        pltpu.make_async_copy(h_hbm.at[pl.ds(0, CH)], rows_s, ssem_a).wait()

    def _complete():
        _drain()
        pltpu.make_async_copy(h_hbm.at[pl.ds(0, CH)], rows_a, gsem_a).wait()

        for j in range(CH // 16):
            w16 = pwsnap[pl.ds(j * 16, 16)]
            for l in range(16):
                wv = w16[l]
                e = j * 16 + l
                for k in range(D // 16):
                    rows_s[e, pl.ds(k * 16, 16)] = (
                        rows_a[e, pl.ds(k * 16, 16)] * wv)
        for j in range(CH // 16):
            dlc_t[1, pl.ds(j * 16, 16)] = dlc_t[0, pl.ds(j * 16, 16)]
        pltpu.async_copy(rows_s, out_sh.at[dlc_t.at[1]], ssem_a, add=True)

    def _flush_shift(q):
        _complete()
        for j in range(CH // 16):
            psnap[pl.ds(j * 16, 16)] = psrc[pl.ds(j * 16, 16)]
            pwsnap[pl.ds(j * 16, 16)] = pw[pl.ds(j * 16, 16)]
            dlc_t[0, pl.ds(j * 16, 16)] = pdl[pl.ds(j * 16, 16)]
        pltpu.async_copy(h_hbm.at[psnap.at[pl.ds(0, CH)]], rows_a, gsem_a)
        for j in range(CH // 16):
            psrc[pl.ds(j * 16, 16)] = psrc[pl.ds(CH + j * 16, 16)]
            pw[pl.ds(j * 16, 16)] = pw[pl.ds(CH + j * 16, 16)]
            pdl[pl.ds(j * 16, 16)] = pdl[pl.ds(CH + j * 16, 16)]
        return q - CH

    def _super(s, qc):
        row0 = sid * NCH + s * 8
        pltpu.sync_copy(src_hbm.at[pl.ds(row0, 8)], src8)
        pltpu.sync_copy(dst_hbm.at[pl.ds(row0, 8)], dst8)

        def _chunk(jj, qc):
            for j in range(CH // 16):
                s16 = src8[jj, pl.ds(j * 16, 16)]
                d16 = dst8[jj, pl.ds(j * 16, 16)]
                a = (plsc.load_gather(as_t, [s16])
                     + plsc.load_gather(ad_t, [d16]))
                a = jnp.where(a > 0, a, a * jnp.float32(0.2))
                w = jnp.exp(a)
                dl = d16 - base
                ok = (dl >= 0) & (dl < HALF)
                plsc.addupdate_scatter(
                    den_t, [jnp.where(ok, dl, jnp.int32(HALF))], w)
                plsc.store_compressed(psrc.at[pl.ds(qc, 16)], s16, mask=ok)
                plsc.store_compressed(pw.at[pl.ds(qc, 16)], w, mask=ok)
                plsc.store_compressed(pdl.at[pl.ds(qc, 16)], dl, mask=ok)
                qc = qc + plsc.all_reduce_population_count(ok)[0]
            return lax.while_loop(lambda q: q >= CH, _flush_shift, qc)

        return lax.fori_loop(0, 8, _chunk, qc)

    qc = lax.fori_loop(0, NCH // 8, _super, jnp.int32(0))

    iota16 = lax.iota(jnp.int32, 16)
    for j in range(CH // 16):
        m = (iota16 + (j * 16)) >= qc
        psrc[pl.ds(j * 16, 16)] = jnp.where(
            m, jnp.int32(NP - 1), psrc[pl.ds(j * 16, 16)])
        pw[pl.ds(j * 16, 16)] = jnp.where(
            m, jnp.float32(0.0), pw[pl.ds(j * 16, 16)])
        pdl[pl.ds(j * 16, 16)] = jnp.where(
            m, jnp.int32(HALF), pdl[pl.ds(j * 16, 16)])
    _flush_shift(jnp.int32(CH))
    _complete()
    _drain()

    plsc.subcore_barrier()

    pltpu.sync_copy(den_t.at[pl.ds(0, HALF)],
                    den_out.at[pl.ds((cid * NS + sid) * HALF, HALF)])
    pltpu.sync_copy(out_sh.at[pl.ds(sid * WB, WB)],
                    un_out.at[cid, pl.ds(sid * WB, WB)])


_sc_call = pl.kernel(
    _sc_body,
    out_type=[
        jax.ShapeDtypeStruct((NC, HALF, D), _f32),
        jax.ShapeDtypeStruct((NC * NS * HALF,), _f32),
    ],
    mesh=plsc.VectorSubcoreMesh(
        core_axis_name="c", subcore_axis_name="s",
        num_cores=NC, num_subcores=NS),
    compiler_params=pltpu.CompilerParams(needs_layout_passes=False),
    scratch_types=[
        pltpu.VMEM((NP,), _f32),
        pltpu.VMEM((NP,), _f32),
        pltpu.VMEM((DEN_P,), _f32),
        pltpu.VMEM((8, CH), jnp.int32),
        pltpu.VMEM((8, CH), jnp.int32),
        pltpu.VMEM((8, CH), jnp.int32),
        pltpu.VMEM((CH, D), _f32),
        pltpu.VMEM((CH, D), _f32),
        pltpu.VMEM((PCAP,), jnp.int32),
        pltpu.VMEM((PCAP,), _f32),
        pltpu.VMEM((PCAP,), jnp.int32),
        pltpu.VMEM((CH,), jnp.int32),
        pltpu.VMEM((CH,), _f32),
        pltpu.VMEM_SHARED((HP, D), _f32),
        pltpu.SemaphoreType.DMA,
        pltpu.SemaphoreType.DMA,
    ],
)


@jax.jit
def kernel(x, edge_index, W1, att_src1, att_dst1, bias1,
           W2, att_src2, att_dst2, bias2):
    x_pad = jnp.zeros((NP, D), _f32).at[:N].set(x)
    ei = edge_index.astype(jnp.int32)
    loop = jnp.arange(N, dtype=jnp.int32)
    pad = jnp.full((EP - NE_REAL,), NP - 1, jnp.int32)
    src = jnp.concatenate([ei[0], loop, pad]).reshape(NS * NCH, CH)
    dst = jnp.concatenate([ei[1], loop, pad]).reshape(NS * NCH, CH)
    att1 = jnp.zeros((8, D), _f32).at[0].set(att_src1).at[1].set(att_dst1)
    att2 = jnp.zeros((8, D), _f32).at[0].set(att_src2).at[1].set(att_dst2)
    b1 = bias1.reshape(1, D)
    b2 = bias2.reshape(1, D)

    h1, asad1 = _k1_call(x_pad, W1, att1)
    un1, den1 = _sc_call(h1, asad1[0], asad1[1], src, dst)
    g2, asad2 = _k2_call(un1, den1.reshape(NC, NS, HALF), b1, W2, att2)
    un2, den2 = _sc_call(g2, asad2[0], asad2[1], src, dst)
    out = _k3_call(un2, den2.reshape(NC, NS, HALF), b2)
    return out[:N]


# speedup vs baseline: 1.1169x; 1.0428x over previous
import jax
import jax.numpy as jnp
from jax import lax
from jax.experimental import pallas as pl
from jax.experimental.pallas import tpu as pltpu
from jax.experimental.pallas import tpu_sc as plsc

N = 10000
E = 320000
D = 128

NP = 10240
NC = 2
NS = 16
HALF = NP // NC
HP = 5248
DEN_P = HALF + 64
CH = 128
NE_REAL = E + N
NCH = (-(-NE_REAL // (NS * CH)) + 7) // 8 * 8
EP = NS * NCH * CH
WB = HALF // NS
PCAP = 384
BR = 1024

_f32 = jnp.float32


def _k1_body(x_ref, w_ref, att_ref, h_ref, asad_ref):
    h = jnp.dot(x_ref[...], w_ref[...], preferred_element_type=_f32)
    h_ref[...] = h
    asad_ref[...] = lax.dot_general(
        att_ref[...], h, (((1,), (1,)), ((), ())), preferred_element_type=_f32)


def _k2_body(un_ref, den_ref, b_ref, w_ref, att_ref, h_ref, asad_ref):
    u = un_ref[0]
    d = jnp.sum(den_ref[0], axis=0) + 1e-30
    h2 = jnp.maximum(u / d[:, None] + b_ref[...], 0.0)
    g = jnp.dot(h2, w_ref[...], preferred_element_type=_f32)
    h_ref[...] = g
    asad_ref[...] = lax.dot_general(
        att_ref[...], g, (((1,), (1,)), ((), ())), preferred_element_type=_f32)


def _k3_body(un_ref, den_ref, b_ref, out_ref):
    u = un_ref[0]
    d = jnp.sum(den_ref[0], axis=0) + 1e-30
    out_ref[...] = u / d[:, None] + b_ref[...]


_HB = HALF // BR

_k1_call = pl.pallas_call(
    _k1_body,
    grid=(NP // BR,),
    in_specs=[
        pl.BlockSpec((BR, D), lambda i: (i, 0)),
        pl.BlockSpec((D, D), lambda i: (0, 0)),
        pl.BlockSpec((8, D), lambda i: (0, 0)),
    ],
    out_specs=[
        pl.BlockSpec((BR, D), lambda i: (i, 0)),
        pl.BlockSpec((8, BR), lambda i: (0, i)),
    ],
    out_shape=[
        jax.ShapeDtypeStruct((NP, D), _f32),
        jax.ShapeDtypeStruct((8, NP), _f32),
    ],
)

_k2_call = pl.pallas_call(
    _k2_body,
    grid=(NP // BR,),
    in_specs=[
        pl.BlockSpec((1, BR, D), lambda i: (i // _HB, i % _HB, 0)),
        pl.BlockSpec((1, NS, BR), lambda i: (i // _HB, 0, i % _HB)),
        pl.BlockSpec((1, D), lambda i: (0, 0)),
        pl.BlockSpec((D, D), lambda i: (0, 0)),
        pl.BlockSpec((8, D), lambda i: (0, 0)),
    ],
    out_specs=[
        pl.BlockSpec((BR, D), lambda i: (i, 0)),
        pl.BlockSpec((8, BR), lambda i: (0, i)),
    ],
    out_shape=[
        jax.ShapeDtypeStruct((NP, D), _f32),
        jax.ShapeDtypeStruct((8, NP), _f32),
    ],
)

_k3_call = pl.pallas_call(
    _k3_body,
    grid=(NP // BR,),
    in_specs=[
        pl.BlockSpec((1, BR, D), lambda i: (i // _HB, i % _HB, 0)),
        pl.BlockSpec((1, NS, BR), lambda i: (i // _HB, 0, i % _HB)),
        pl.BlockSpec((1, D), lambda i: (0, 0)),
    ],
    out_specs=pl.BlockSpec((BR, D), lambda i: (i, 0)),
    out_shape=jax.ShapeDtypeStruct((NP, D), _f32),
)


def _sc_body(h_hbm, as_hbm, ad_hbm, src_hbm, dst_hbm, un_out, den_out,
             as_t, ad_t, den_t, src8, dst8, dlc_t, rows_a,
             psrc, pw, pdl, out_sh, gsem_a, ssem_a):
    cid = lax.axis_index("c")
    sid = lax.axis_index("s")

    z16 = jnp.zeros((16,), _f32)

    def _zrow(r, _):
        for k in range(D // 16):
            rows_a[r, pl.ds(k * 16, 16)] = z16
        return _
    lax.fori_loop(0, CH, _zrow, None)
    for q in range(-(-(HP // NS)) // CH + 1):
        row0 = jnp.minimum(sid * (HP // NS) + q * CH, HP - CH)
        pltpu.sync_copy(rows_a, out_sh.at[pl.ds(row0, CH)])

    def _zden(i, _):
        den_t[pl.ds(i * 16, 16)] = z16
        return _
    lax.fori_loop(0, DEN_P // 16, _zden, None)

    pltpu.sync_copy(as_hbm, as_t)
    pltpu.sync_copy(ad_hbm, ad_t)

    pltpu.async_copy(rows_a, out_sh.at[pl.ds(HALF, CH)], ssem_a)

    plsc.subcore_barrier()

    base = cid * HALF

    def _drain():
        pltpu.make_async_copy(h_hbm.at[pl.ds(0, CH)], rows_a, ssem_a).wait()

    def _flush_body():
        _drain()
        for j in range(CH // 16):
            dlc_t[0, pl.ds(j * 16, 16)] = pdl[pl.ds(j * 16, 16)]
        pltpu.async_copy(
            h_hbm.at[psrc.at[pl.ds(0, CH)]], rows_a, gsem_a).wait()
        for j in range(CH // 16):
            w16 = pw[pl.ds(j * 16, 16)]
            for l in range(16):
                e = j * 16 + l
                wv = w16[l]
                for k in range(D // 16):
                    rows_a[e, pl.ds(k * 16, 16)] = (
                        rows_a[e, pl.ds(k * 16, 16)] * wv)
        pltpu.async_copy(rows_a, out_sh.at[dlc_t.at[0]], ssem_a, add=True)

    def _flush_shift(q):
        _flush_body()
        for j in range(CH // 16):
            psrc[pl.ds(j * 16, 16)] = psrc[pl.ds(CH + j * 16, 16)]
            pw[pl.ds(j * 16, 16)] = pw[pl.ds(CH + j * 16, 16)]
            pdl[pl.ds(j * 16, 16)] = pdl[pl.ds(CH + j * 16, 16)]
        return q - CH

    def _super(s, qc):
        row0 = sid * NCH + s * 8
        pltpu.sync_copy(src_hbm.at[pl.ds(row0, 8)], src8)
        pltpu.sync_copy(dst_hbm.at[pl.ds(row0, 8)], dst8)

        def _chunk(jj, qc):
            for j in range(CH // 16):
                s16 = src8[jj, pl.ds(j * 16, 16)]
                d16 = dst8[jj, pl.ds(j * 16, 16)]
                a = (plsc.load_gather(as_t, [s16])
                     + plsc.load_gather(ad_t, [d16]))
                a = jnp.where(a > 0, a, a * jnp.float32(0.2))
                w = jnp.exp(a)
                dl = d16 - base
                ok = (dl >= 0) & (dl < HALF)
                plsc.addupdate_scatter(
                    den_t, [jnp.where(ok, dl, jnp.int32(HALF))], w)
                plsc.store_compressed(psrc.at[pl.ds(qc, 16)], s16, mask=ok)
                plsc.store_compressed(pw.at[pl.ds(qc, 16)], w, mask=ok)
                plsc.store_compressed(pdl.at[pl.ds(qc, 16)], dl, mask=ok)
                qc = qc + plsc.all_reduce_population_count(ok)[0]
            return lax.while_loop(lambda q: q >= CH, _flush_shift, qc)

        return lax.fori_loop(0, 8, _chunk, qc)

    qc = lax.fori_loop(0, NCH // 8, _super, jnp.int32(0))

    iota16 = lax.iota(jnp.int32, 16)
    for j in range(CH // 16):
        m = (iota16 + (j * 16)) >= qc
        psrc[pl.ds(j * 16, 16)] = jnp.where(
            m, jnp.int32(NP - 1), psrc[pl.ds(j * 16, 16)])
        pw[pl.ds(j * 16, 16)] = jnp.where(
            m, jnp.float32(0.0), pw[pl.ds(j * 16, 16)])
        pdl[pl.ds(j * 16, 16)] = jnp.where(
            m, jnp.int32(HALF), pdl[pl.ds(j * 16, 16)])
    _flush_body()
    _drain()

    plsc.subcore_barrier()

    pltpu.sync_copy(den_t.at[pl.ds(0, HALF)],
                    den_out.at[pl.ds((cid * NS + sid) * HALF, HALF)])
    pltpu.sync_copy(out_sh.at[pl.ds(sid * WB, WB)],
                    un_out.at[cid, pl.ds(sid * WB, WB)])


_sc_call = pl.kernel(
    _sc_body,
    out_type=[
        jax.ShapeDtypeStruct((NC, HALF, D), _f32),
        jax.ShapeDtypeStruct((NC * NS * HALF,), _f32),
    ],
    mesh=plsc.VectorSubcoreMesh(
        core_axis_name="c", subcore_axis_name="s",
        num_cores=NC, num_subcores=NS),
    compiler_params=pltpu.CompilerParams(needs_layout_passes=False),
    scratch_types=[
        pltpu.VMEM((NP,), _f32),
        pltpu.VMEM((NP,), _f32),
        pltpu.VMEM((DEN_P,), _f32),
        pltpu.VMEM((8, CH), jnp.int32),
        pltpu.VMEM((8, CH), jnp.int32),
        pltpu.VMEM((8, CH), jnp.int32),
        pltpu.VMEM((CH, D), _f32),
        pltpu.VMEM((PCAP,), jnp.int32),
        pltpu.VMEM((PCAP,), _f32),
        pltpu.VMEM((PCAP,), jnp.int32),
        pltpu.VMEM_SHARED((HP, D), _f32),
        pltpu.SemaphoreType.DMA,
        pltpu.SemaphoreType.DMA,
    ],
)


@jax.jit
def kernel(x, edge_index, W1, att_src1, att_dst1, bias1,
           W2, att_src2, att_dst2, bias2):
    x_pad = jnp.zeros((NP, D), _f32).at[:N].set(x)
    ei = edge_index.astype(jnp.int32)
    loop = jnp.arange(N, dtype=jnp.int32)
    pad = jnp.full((EP - NE_REAL,), NP - 1, jnp.int32)
    src = jnp.concatenate([ei[0], loop, pad]).reshape(NS * NCH, CH)
    dst = jnp.concatenate([ei[1], loop, pad]).reshape(NS * NCH, CH)
    att1 = jnp.zeros((8, D), _f32).at[0].set(att_src1).at[1].set(att_dst1)
    att2 = jnp.zeros((8, D), _f32).at[0].set(att_src2).at[1].set(att_dst2)
    b1 = bias1.reshape(1, D)
    b2 = bias2.reshape(1, D)

    h1, asad1 = _k1_call(x_pad, W1, att1)
    un1, den1 = _sc_call(h1, asad1[0], asad1[1], src, dst)
    g2, asad2 = _k2_call(un1, den1.reshape(NC, NS, HALF), b1, W2, att2)
    un2, den2 = _sc_call(g2, asad2[0], asad2[1], src, dst)
    out = _k3_call(un2, den2.reshape(NC, NS, HALF), b2)
    return out[:N]
